# baseline (speedup 1.0000x reference)
import functools

import jax
import jax.numpy as jnp
from jax import lax
from jax.experimental import pallas as pl
from jax.experimental.pallas import tpu as pltpu
from jax.experimental.pallas import tpu_sc as plsc

_VOCAB = 1_000_000
_DIM = 64
_BATCH = 16384
_NEG = 20

_NC = 2
_NS = 16
_NW = _NC * _NS
_NB = _BATCH // _NW
_CB = 32
_NCHUNK = _NB // _CB
_IDXROW = 128
_NEG_GATHERS = _CB * _NEG // _IDXROW
_NVREG = _DIM // 16


def _sc_scores(emb_v, emb_u, center_idx, target_idx, neg_idx):
    mesh = plsc.VectorSubcoreMesh(core_axis_name="c", subcore_axis_name="s")

    @functools.partial(
        pl.kernel,
        mesh=mesh,
        compiler_params=pltpu.CompilerParams(
            needs_layout_passes=False, use_tc_tiling_on_sc=True),
        out_type=(
            jax.ShapeDtypeStruct((_BATCH,), jnp.float32),
            jax.ShapeDtypeStruct((_BATCH,), jnp.float32),
        ),
        scratch_types=[
            pltpu.VMEM((_NB,), jnp.int32),
            pltpu.VMEM((_NB,), jnp.int32),
            pltpu.VMEM((_NB * _NEG,), jnp.int32),
            pltpu.VMEM((_CB, 2 * _DIM), jnp.float32),
            pltpu.VMEM((_CB, 2 * _DIM), jnp.float32),
            pltpu.VMEM((_CB * _NEG, 2 * _DIM), jnp.float32),
            pltpu.VMEM((_NB,), jnp.float32),
            pltpu.VMEM((_NB,), jnp.float32),
            pltpu.SemaphoreType.DMA,
        ],
    )
    def scores(v_hbm, u_hbm, cidx_hbm, tidx_hbm, nidx_hbm,
               pos_hbm, negs_hbm,
               cidx, tidx, nidx, vrows, trows, nrows, posb, negb, sem):
        wid = lax.axis_index("s") * _NC + lax.axis_index("c")
        base = wid * _NB
        last_lane = lax.iota(jnp.int32, 16) == 15
        pltpu.sync_copy(cidx_hbm.at[pl.ds(base, _NB)], cidx)
        pltpu.sync_copy(tidx_hbm.at[pl.ds(base, _NB)], tidx)
        pltpu.sync_copy(nidx_hbm.at[pl.ds(base * _NEG, _NB * _NEG)], nidx)

        for c in range(_NCHUNK):
            cb = c * _CB
            cps = [
                pltpu.async_copy(v_hbm.at[cidx.at[pl.ds(cb, _CB)]],
                                 vrows, sem),
                pltpu.async_copy(u_hbm.at[tidx.at[pl.ds(cb, _CB)]],
                                 trows, sem),
            ]
            for g in range(_NEG_GATHERS):
                cps.append(pltpu.async_copy(
                    u_hbm.at[nidx.at[pl.ds(cb * _NEG + g * _IDXROW, _IDXROW)]],
                    nrows.at[pl.ds(g * _IDXROW, _IDXROW)],
                    sem))
            for cp in cps:
                cp.wait()

            def body(b, carry, cb=cb):
                v = [vrows[b, pl.ds(16 * j, 16)] for j in range(_NVREG)]
                t = [trows[b, pl.ds(16 * j, 16)] for j in range(_NVREG)]
                r0 = b * _NEG
                acc = [nrows[r0, pl.ds(16 * j, 16)] for j in range(_NVREG)]
                for k in range(1, _NEG):
                    for j in range(_NVREG):
                        acc[j] = acc[j] + nrows[r0 + k, pl.ds(16 * j, 16)]
                pos_l = t[0] * v[0]
                neg_l = acc[0] * v[0]
                for j in range(1, _NVREG):
                    pos_l = pos_l + t[j] * v[j]
                    neg_l = neg_l + acc[j] * v[j]
                out_idx = jnp.full((16,), cb + b, jnp.int32)
                plsc.store_scatter(posb, [out_idx], plsc.cumsum(pos_l),
                                   mask=last_lane)
                plsc.store_scatter(negb, [out_idx], plsc.cumsum(neg_l),
                                   mask=last_lane)
                return carry

            lax.fori_loop(0, _CB, body, 0)

        pltpu.sync_copy(posb, pos_hbm.at[pl.ds(base, _NB)])
        pltpu.sync_copy(negb, negs_hbm.at[pl.ds(base, _NB)])

    return scores(emb_v, emb_u, center_idx, target_idx, neg_idx)


def _loss_tc(pos, neg):
    def body(pos_ref, neg_ref, out_ref):
        loss = (jax.nn.log_sigmoid(pos_ref[...])
                + jax.nn.log_sigmoid(-neg_ref[...]))
        out_ref[0, 0] = -jnp.sum(loss) / _BATCH

    return pl.pallas_call(
        body,
        out_shape=jax.ShapeDtypeStruct((1, 1), jnp.float32),
        out_specs=pl.BlockSpec(memory_space=pltpu.SMEM),
    )(pos.reshape(128, 128), neg.reshape(128, 128))


def kernel(center_words, target_words, negative_words, embedding_v, embedding_u):
    c = center_words.reshape(-1).astype(jnp.int32) >> 1
    t = target_words.reshape(-1).astype(jnp.int32) >> 1
    n = negative_words.reshape(-1).astype(jnp.int32) >> 1
    vt = embedding_v.reshape(_VOCAB // 2, 2 * _DIM)
    ut = embedding_u.reshape(_VOCAB // 2, 2 * _DIM)
    pos, neg = _sc_scores(vt, ut, c, t, n)
    return _loss_tc(pos, neg)[0, 0]


# speedup vs baseline: 1.0564x; 1.0564x over previous
import functools

import jax
import jax.numpy as jnp
from jax import lax
from jax.experimental import pallas as pl
from jax.experimental.pallas import tpu as pltpu
from jax.experimental.pallas import tpu_sc as plsc

_VOCAB = 1_000_000
_DIM = 64
_BATCH = 16384
_NEG = 20

_NC = 2
_NS = 16
_NW = _NC * _NS
_NB = _BATCH // _NW
_CB = 32
_NCHUNK = _NB // _CB
_IDXROW = 128
_NEG_GATHERS = _CB * _NEG // _IDXROW
_NVREG = _DIM // 16


def _sc_scores(emb_v, emb_u, center_idx, target_idx, neg_idx):
    mesh = plsc.VectorSubcoreMesh(core_axis_name="c", subcore_axis_name="s")

    @functools.partial(
        pl.kernel,
        mesh=mesh,
        compiler_params=pltpu.CompilerParams(
            needs_layout_passes=False, use_tc_tiling_on_sc=False),
        out_type=(
            jax.ShapeDtypeStruct((_BATCH,), jnp.float32),
            jax.ShapeDtypeStruct((_BATCH,), jnp.float32),
        ),
        scratch_types=[
            pltpu.VMEM((_NB,), jnp.int32),
            pltpu.VMEM((_NB,), jnp.int32),
            pltpu.VMEM((_NB * _NEG,), jnp.int32),
            pltpu.VMEM((2, _CB, _DIM), jnp.float32),
            pltpu.VMEM((2, _CB, _DIM), jnp.float32),
            pltpu.VMEM((2, _CB * _NEG, _DIM), jnp.float32),
            pltpu.VMEM((_NB,), jnp.float32),
            pltpu.VMEM((_NB,), jnp.float32),
            pltpu.SemaphoreType.DMA,
            pltpu.SemaphoreType.DMA,
        ],
    )
    def scores(v_hbm, u_hbm, cidx_hbm, tidx_hbm, nidx_hbm,
               pos_hbm, negs_hbm,
               cidx, tidx, nidx, vrows, trows, nrows, posb, negb,
               sem0, sem1):
        wid = lax.axis_index("s") * _NC + lax.axis_index("c")
        base = wid * _NB
        last_lane = lax.iota(jnp.int32, 16) == 15
        pltpu.sync_copy(cidx_hbm.at[pl.ds(base, _NB)], cidx)
        pltpu.sync_copy(tidx_hbm.at[pl.ds(base, _NB)], tidx)
        pltpu.sync_copy(nidx_hbm.at[pl.ds(base * _NEG, _NB * _NEG)], nidx)

        def issue(c, p, sem):
            cb = c * _CB
            cps = [
                pltpu.async_copy(v_hbm.at[cidx.at[pl.ds(cb, _CB)]],
                                 vrows.at[p], sem),
                pltpu.async_copy(u_hbm.at[tidx.at[pl.ds(cb, _CB)]],
                                 trows.at[p], sem),
            ]
            for g in range(_NEG_GATHERS):
                cps.append(pltpu.async_copy(
                    u_hbm.at[nidx.at[pl.ds(cb * _NEG + g * _IDXROW, _IDXROW)]],
                    nrows.at[p, pl.ds(g * _IDXROW, _IDXROW)],
                    sem))
            return cps

        sems = (sem0, sem1)
        pending = issue(0, 0, sem0)
        for c in range(_NCHUNK):
            p = c % 2
            nxt = (issue(c + 1, 1 - p, sems[1 - p])
                   if c + 1 < _NCHUNK else [])
            for cp in pending:
                cp.wait()
            cb = c * _CB

            def body(b, carry, p=p, cb=cb):
                v = [vrows[p, b, pl.ds(16 * j, 16)] for j in range(_NVREG)]
                t = [trows[p, b, pl.ds(16 * j, 16)] for j in range(_NVREG)]
                r0 = b * _NEG
                acc = [nrows[p, r0, pl.ds(16 * j, 16)]
                       for j in range(_NVREG)]
                for k in range(1, _NEG):
                    for j in range(_NVREG):
                        acc[j] = acc[j] + nrows[p, r0 + k, pl.ds(16 * j, 16)]
                pos_l = t[0] * v[0]
                neg_l = acc[0] * v[0]
                for j in range(1, _NVREG):
                    pos_l = pos_l + t[j] * v[j]
                    neg_l = neg_l + acc[j] * v[j]
                out_idx = jnp.full((16,), cb + b, jnp.int32)
                plsc.store_scatter(posb, [out_idx], plsc.cumsum(pos_l),
                                   mask=last_lane)
                plsc.store_scatter(negb, [out_idx], plsc.cumsum(neg_l),
                                   mask=last_lane)
                return carry

            lax.fori_loop(0, _CB, body, 0)
            pending = nxt

        pltpu.sync_copy(posb, pos_hbm.at[pl.ds(base, _NB)])
        pltpu.sync_copy(negb, negs_hbm.at[pl.ds(base, _NB)])

    return scores(emb_v, emb_u, center_idx, target_idx, neg_idx)


def _loss_tc(pos, neg):
    def body(pos_ref, neg_ref, out_ref):
        loss = (jax.nn.log_sigmoid(pos_ref[...])
                + jax.nn.log_sigmoid(-neg_ref[...]))
        out_ref[0, 0] = -jnp.sum(loss) / _BATCH

    return pl.pallas_call(
        body,
        out_shape=jax.ShapeDtypeStruct((1, 1), jnp.float32),
        out_specs=pl.BlockSpec(memory_space=pltpu.SMEM),
    )(pos.reshape(128, 128), neg.reshape(128, 128))


def kernel(center_words, target_words, negative_words, embedding_v, embedding_u):
    c = center_words.reshape(-1).astype(jnp.int32)
    t = target_words.reshape(-1).astype(jnp.int32)
    n = negative_words.reshape(-1).astype(jnp.int32)
    pos, neg = _sc_scores(embedding_v, embedding_u, c, t, n)
    return _loss_tc(pos, neg)[0, 0]
